# baseline (speedup 1.0000x reference)
import math

import jax
import jax.numpy as jnp
from jax import lax
from jax.experimental import pallas as pl
from jax.experimental.pallas import tpu as pltpu
from jax.experimental.pallas import tpu_sc as plsc

VOCAB = 32000
SMOOTH = 0.1
CONF = 1.0 - SMOOTH
EPS = SMOOTH / (VOCAB - 2)
ROW_CONST = float((VOCAB - 2) * EPS * math.log(EPS) + CONF * math.log(CONF))

LANES = 16
NUM_WORKERS = 32
N_ROWS = 4096
ROW_BLK = 128
TC_ROWS = 3584
SPW = (N_ROWS - TC_ROWS) // NUM_WORKERS


def _row_sum_inner(buf):
    def body(k, c):
        a0, a1 = c
        o = k * 128
        for u in range(4):
            a0 = a0 + buf[pl.ds(o + u * 32, LANES)]
            a1 = a1 + buf[pl.ds(o + u * 32 + LANES, LANES)]
        return a0, a1
    z = jnp.zeros((LANES,), jnp.float32)
    a0, a1 = lax.fori_loop(0, VOCAB // 128, body, (z, z))
    return a0 + a1


def _sc_body(x_hbm, y_hbm, out_hbm, y_v, row0_v, row1_v, acc_v,
             sem_r0, sem_r1):
    wid = lax.axis_index("s") * 2 + lax.axis_index("c")
    rs_base = TC_ROWS + wid * SPW

    pltpu.sync_copy(y_hbm.at[pl.ds(rs_base, SPW)], y_v)

    bufs = (row0_v, row1_v)
    sems = (sem_r0, sem_r1)

    def fire(r, b):
        return pltpu.async_copy(x_hbm.at[rs_base + r], bufs[b], sems[b])

    fire(0, 0)
    if SPW > 1:
        fire(1, 1)

    iota = lax.iota(jnp.int32, LANES)
    zeros_f = jnp.zeros((LANES,), jnp.float32)
    c_vec = jnp.where(iota == 0, jnp.float32(ROW_CONST), zeros_f)
    acc = zeros_f
    for r in range(SPW):
        b = r % 2
        pltpu.make_async_copy(x_hbm.at[rs_base + r], bufs[b], sems[b]).wait()
        row_acc = _row_sum_inner(bufs[b])
        yv = y_v[pl.ds((r // LANES) * LANES, LANES)]
        yr = yv[r % LANES]
        mf = jnp.where(yr != 0, jnp.float32(1.0), jnp.float32(0.0))
        gv = bufs[b][pl.ds((yr // LANES) * LANES, LANES)]
        gsel = jnp.where(iota == yr % LANES, gv, zeros_f)
        x0v = bufs[b][pl.ds(0, LANES)]
        x0sel = jnp.where(iota == 0, x0v, zeros_f)
        acc = acc + mf * (c_vec + EPS * x0sel + (EPS - CONF) * gsel
                          - EPS * row_acc)
        if r + 2 < SPW:
            fire(r + 2, b)

    acc_v[pl.ds(0, LANES)] = acc
    for t in range(1, 128 // LANES):
        acc_v[pl.ds(t * LANES, LANES)] = zeros_f
    pltpu.sync_copy(acc_v, out_hbm.at[wid])


def _sc_part(x, y32):
    mesh = plsc.VectorSubcoreMesh(core_axis_name="c", subcore_axis_name="s",
                                  num_cores=2, num_subcores=16)
    kern = pl.kernel(
        _sc_body,
        out_type=jax.ShapeDtypeStruct((NUM_WORKERS, 128), jnp.float32),
        mesh=mesh,
        scratch_types=[
            pltpu.VMEM((SPW,), jnp.int32),
            pltpu.VMEM((VOCAB,), jnp.float32),
            pltpu.VMEM((VOCAB,), jnp.float32),
            pltpu.VMEM((128,), jnp.float32),
            pltpu.SemaphoreType.DMA,
            pltpu.SemaphoreType.DMA,
        ],
    )
    return kern(x, y32)


def _tc_body(x_ref, y_ref, ys_ref, o_ref, acc_ref):
    i = pl.program_id(0)
    n = pl.num_programs(0)

    @pl.when(i == 0)
    def _():
        acc_ref[0, 0] = 0.0

    row_sums = jnp.sum(x_ref[...], axis=1)
    liota = lax.broadcasted_iota(jnp.int32, (ROW_BLK, 128), 1)
    x0 = jnp.sum(jnp.where(liota == 0, x_ref[:, :128], 0.0), axis=1)
    mask = y_ref[0, 0, :] != 0
    vec_part = jnp.sum(
        jnp.where(mask, ROW_CONST + EPS * x0 - EPS * row_sums, 0.0))

    siota = lax.broadcasted_iota(jnp.int32, (8, 128), 0)
    tiota = lax.broadcasted_iota(jnp.int32, (8, 128), 1)

    def body(r, g_acc):
        yr = ys_ref[i * ROW_BLK + r]
        rbase = pl.multiple_of((r // 8) * 8, 8)
        cbase = pl.multiple_of((yr // 128) * 128, 128)
        chunk = x_ref[pl.ds(rbase, 8), pl.ds(cbase, 128)]
        hit = (siota == r % 8) & (tiota == yr % 128) & (yr != 0)
        return g_acc + jnp.where(hit, chunk, 0.0)

    g_acc = lax.fori_loop(0, ROW_BLK, body, jnp.zeros((8, 128), jnp.float32))
    acc_ref[0, 0] += vec_part + (EPS - CONF) * jnp.sum(g_acc)

    @pl.when(i == n - 1)
    def _():
        o_ref[0, 0] = acc_ref[0, 0]


def kernel(x, y, normalizer):
    n, vocab = x.shape
    y32 = y.astype(jnp.int32)

    sc_out = _sc_part(x, y32)

    grid = TC_ROWS // ROW_BLK
    y3 = y32[:TC_ROWS].reshape(grid, 1, ROW_BLK)

    tc_out = pl.pallas_call(
        _tc_body,
        grid=(grid,),
        in_specs=[
            pl.BlockSpec((ROW_BLK, vocab), lambda i: (i, 0)),
            pl.BlockSpec((1, 1, ROW_BLK), lambda i: (i, 0, 0)),
            pl.BlockSpec(memory_space=pltpu.SMEM),
        ],
        out_specs=pl.BlockSpec(memory_space=pltpu.SMEM),
        out_shape=jax.ShapeDtypeStruct((1, 1), jnp.float32),
        scratch_shapes=[pltpu.SMEM((1, 1), jnp.float32)],
    )(x, y3, y32)

    return (jnp.sum(sc_out) + tc_out[0, 0]) / normalizer


# speedup vs baseline: 1.0140x; 1.0140x over previous
import math

import jax
import jax.numpy as jnp
from jax import lax
from jax.experimental import pallas as pl
from jax.experimental.pallas import tpu as pltpu
from jax.experimental.pallas import tpu_sc as plsc

VOCAB = 32000
SMOOTH = 0.1
CONF = 1.0 - SMOOTH
EPS = SMOOTH / (VOCAB - 2)
ROW_CONST = float((VOCAB - 2) * EPS * math.log(EPS) + CONF * math.log(CONF))

LANES = 16
NUM_WORKERS = 32
N_ROWS = 4096
ROW_BLK = 128


def _sc_body(y_hbm, out_hbm, y_v, acc_v, *, rows_per_worker):
    wid = lax.axis_index("s") * 2 + lax.axis_index("c")
    base = wid * rows_per_worker

    pltpu.sync_copy(y_hbm.at[pl.ds(base, rows_per_worker)], y_v)

    zeros_f = jnp.zeros((LANES,), jnp.float32)
    acc = zeros_f
    for j in range(rows_per_worker // LANES):
        yv = y_v[pl.ds(j * LANES, LANES)]
        acc = acc + jnp.where(yv != 0, jnp.float32(ROW_CONST), zeros_f)

    acc_v[pl.ds(0, LANES)] = acc
    for t in range(1, 128 // LANES):
        acc_v[pl.ds(t * LANES, LANES)] = zeros_f
    pltpu.sync_copy(acc_v, out_hbm.at[wid])


def _sc_const_part(y32):
    mesh = plsc.VectorSubcoreMesh(core_axis_name="c", subcore_axis_name="s",
                                  num_cores=2, num_subcores=16)
    rpw = N_ROWS // NUM_WORKERS
    kern = pl.kernel(
        lambda y_hbm, out_hbm, y_v, acc_v: _sc_body(
            y_hbm, out_hbm, y_v, acc_v, rows_per_worker=rpw),
        out_type=jax.ShapeDtypeStruct((NUM_WORKERS, 128), jnp.float32),
        mesh=mesh,
        scratch_types=[
            pltpu.VMEM((rpw,), jnp.int32),
            pltpu.VMEM((128,), jnp.float32),
        ],
    )
    return kern(y32)


def _tc_body(x_ref, y_ref, ys_ref, o_ref, acc_ref):
    i = pl.program_id(0)
    n = pl.num_programs(0)

    @pl.when(i == 0)
    def _():
        acc_ref[0, 0] = 0.0

    row_sums = jnp.sum(x_ref[...], axis=1)
    liota = lax.broadcasted_iota(jnp.int32, (ROW_BLK, 128), 1)
    x0 = jnp.sum(jnp.where(liota == 0, x_ref[:, :128], 0.0), axis=1)
    mask = y_ref[0, 0, :] != 0
    vec_part = jnp.sum(jnp.where(mask, EPS * x0 - EPS * row_sums, 0.0))

    siota = lax.broadcasted_iota(jnp.int32, (8, 128), 0)
    tiota = lax.broadcasted_iota(jnp.int32, (8, 128), 1)

    def body(r, g_acc):
        yr = ys_ref[i * ROW_BLK + r]
        rbase = pl.multiple_of((r // 8) * 8, 8)
        cbase = pl.multiple_of((yr // 128) * 128, 128)
        chunk = x_ref[pl.ds(rbase, 8), pl.ds(cbase, 128)]
        hit = (siota == r % 8) & (tiota == yr % 128) & (yr != 0)
        return g_acc + jnp.where(hit, chunk, 0.0)

    g_acc = lax.fori_loop(0, ROW_BLK, body, jnp.zeros((8, 128), jnp.float32))
    acc_ref[0, 0] += vec_part + (EPS - CONF) * jnp.sum(g_acc)

    @pl.when(i == n - 1)
    def _():
        o_ref[0, 0] = acc_ref[0, 0]


def kernel(x, y, normalizer):
    n, vocab = x.shape
    y32 = y.astype(jnp.int32)

    grid = n // ROW_BLK
    y3 = y32.reshape(grid, 1, ROW_BLK)

    tc_out = pl.pallas_call(
        _tc_body,
        grid=(grid,),
        in_specs=[
            pl.BlockSpec((ROW_BLK, vocab), lambda i: (i, 0)),
            pl.BlockSpec((1, 1, ROW_BLK), lambda i: (i, 0, 0)),
            pl.BlockSpec(memory_space=pltpu.SMEM),
        ],
        out_specs=pl.BlockSpec(memory_space=pltpu.SMEM),
        out_shape=jax.ShapeDtypeStruct((1, 1), jnp.float32),
        scratch_shapes=[pltpu.SMEM((1, 1), jnp.float32)],
    )(x, y3, y32)

    sc_out = _sc_const_part(y32)
    return (jnp.sum(sc_out) + tc_out[0, 0]) / normalizer


# speedup vs baseline: 1.0430x; 1.0286x over previous
import math

import jax
import jax.numpy as jnp
from jax import lax
from jax.experimental import pallas as pl
from jax.experimental.pallas import tpu as pltpu
from jax.experimental.pallas import tpu_sc as plsc

VOCAB = 32000
SMOOTH = 0.1
CONF = 1.0 - SMOOTH
EPS = SMOOTH / (VOCAB - 2)
ROW_CONST = float((VOCAB - 2) * EPS * math.log(EPS) + CONF * math.log(CONF))

LANES = 16
NUM_WORKERS = 32
N_ROWS = 4096
ROW_BLK = 128
RPW = N_ROWS // NUM_WORKERS


def _sc_body(x_hbm, y_hbm, out_hbm, y_v, g_v, x0_v, acc_v, sem_g, sem_0):
    wid = lax.axis_index("s") * 2 + lax.axis_index("c")
    base = wid * RPW

    pltpu.sync_copy(y_hbm.at[pl.ds(base, RPW)], y_v)

    iota = lax.iota(jnp.int32, LANES)
    zeros_f = jnp.zeros((LANES,), jnp.float32)
    c_vec = jnp.where(iota == 0, jnp.float32(ROW_CONST), zeros_f)
    acc = zeros_f
    for j in range(RPW // LANES):
        yv = y_v[pl.ds(j * LANES, LANES)]
        for u in range(LANES):
            r = j * LANES + u
            yr = yv[u]
            pltpu.async_copy(
                x_hbm.at[base + r, pl.ds((yr // LANES) * LANES, LANES)],
                g_v.at[u], sem_g)
            pltpu.async_copy(x_hbm.at[base + r, pl.ds(0, LANES)],
                             x0_v.at[u], sem_0)
        for u in range(LANES):
            yr = yv[u]
            pltpu.make_async_copy(
                x_hbm.at[base, pl.ds(0, LANES)], g_v.at[u], sem_g).wait()
            pltpu.make_async_copy(
                x_hbm.at[base, pl.ds(0, LANES)], x0_v.at[u], sem_0).wait()
            mf = jnp.where(yr != 0, jnp.float32(1.0), jnp.float32(0.0))
            gsel = jnp.where(iota == yr % LANES, g_v[u], zeros_f)
            x0sel = jnp.where(iota == 0, x0_v[u], zeros_f)
            acc = acc + mf * (c_vec + EPS * x0sel + (EPS - CONF) * gsel)

    acc_v[pl.ds(0, LANES)] = acc
    for t in range(1, 128 // LANES):
        acc_v[pl.ds(t * LANES, LANES)] = zeros_f
    pltpu.sync_copy(acc_v, out_hbm.at[wid])


def _sc_gather_part(x, y32):
    mesh = plsc.VectorSubcoreMesh(core_axis_name="c", subcore_axis_name="s",
                                  num_cores=2, num_subcores=16)
    kern = pl.kernel(
        _sc_body,
        out_type=jax.ShapeDtypeStruct((NUM_WORKERS, 128), jnp.float32),
        mesh=mesh,
        scratch_types=[
            pltpu.VMEM((RPW,), jnp.int32),
            pltpu.VMEM((LANES, LANES), jnp.float32),
            pltpu.VMEM((LANES, LANES), jnp.float32),
            pltpu.VMEM((128,), jnp.float32),
            pltpu.SemaphoreType.DMA,
            pltpu.SemaphoreType.DMA,
        ],
    )
    return kern(x, y32)


def _tc_body(x_ref, y_ref, o_ref, acc_ref):
    i = pl.program_id(0)
    n = pl.num_programs(0)

    @pl.when(i == 0)
    def _():
        acc_ref[0, 0] = 0.0

    row_sums = jnp.sum(x_ref[...], axis=1)
    mask = y_ref[0, 0, :] != 0
    acc_ref[0, 0] += jnp.sum(jnp.where(mask, row_sums, 0.0))

    @pl.when(i == n - 1)
    def _():
        o_ref[0, 0] = acc_ref[0, 0]


def kernel(x, y, normalizer):
    n, vocab = x.shape
    y32 = y.astype(jnp.int32)

    sc_out = _sc_gather_part(x, y32)

    grid = n // ROW_BLK
    y3 = y32.reshape(grid, 1, ROW_BLK)

    tc_out = pl.pallas_call(
        _tc_body,
        grid=(grid,),
        in_specs=[
            pl.BlockSpec((ROW_BLK, vocab), lambda i: (i, 0)),
            pl.BlockSpec((1, 1, ROW_BLK), lambda i: (i, 0, 0)),
        ],
        out_specs=pl.BlockSpec(memory_space=pltpu.SMEM),
        out_shape=jax.ShapeDtypeStruct((1, 1), jnp.float32),
        scratch_shapes=[pltpu.SMEM((1, 1), jnp.float32)],
    )(x, y3)

    return (jnp.sum(sc_out) - EPS * tc_out[0, 0]) / normalizer
